# Initial kernel scaffold; baseline (speedup 1.0000x reference)
import jax
import jax.numpy as jnp
from jax.experimental import pallas as pl


def kernel(x, edge_index, batch, Wl0, bl0, Wr0, Wl1, bl1, Wr1, Wl2, bl2, Wr2, Wl3, bl3, Wr3, head_W, head_b):
    raise NotImplementedError("write your pallas kernel here")


# speedup vs baseline: 4.5480x; 4.5480x over previous
import functools

import jax
import jax.numpy as jnp
from jax import lax
from jax.experimental import pallas as pl
from jax.experimental.pallas import tpu as pltpu
from jax.experimental.pallas import tpu_sc as plsc

N = 10000
E = 160000
H = 256
G = 64
HALF = 128
NC = 2
NS = 16
EPT = E // NS
CH = 80
NCH = EPT // CH
RPT = 624
TAIL = N - NS * RPT
BN = 1000
NB = N // BN

def _sc_degree_body(dst_hbm, ones_hbm, zeros_hbm, deg_hbm, dst_v, ones_v, deg_sh):
    c = lax.axis_index("c")
    s = lax.axis_index("s")
    pltpu.sync_copy(zeros_hbm, deg_sh.at[pl.ds(s * RPT, RPT)])

    @pl.when(s == 0)
    def _():
        pltpu.sync_copy(
            zeros_hbm.at[pl.ds(0, TAIL)], deg_sh.at[pl.ds(NS * RPT, TAIL)]
        )

    pltpu.sync_copy(dst_hbm.at[s], dst_v)
    pltpu.sync_copy(ones_hbm, ones_v)
    plsc.subcore_barrier()

    @pl.loop(0, NCH // 2)
    def _(j):
        pltpu.sync_copy(ones_v, deg_sh.at[dst_v.at[2 * j + c]], add=True)

    @pl.when(c == 0)
    def _():
        pltpu.sync_copy(ones_v, deg_sh.at[dst_v.at[NCH - 1]], add=True)

    plsc.subcore_barrier()
    pltpu.sync_copy(
        deg_sh.at[pl.ds(s * RPT, RPT)],
        deg_hbm.at[pl.ds(c * N + s * RPT, RPT)],
    )

    @pl.when(s == 0)
    def _():
        pltpu.sync_copy(
            deg_sh.at[pl.ds(NS * RPT, TAIL)],
            deg_hbm.at[pl.ds(c * N + NS * RPT, TAIL)],
        )


def _sc_agg_body(hflat_hbm, src2_hbm, dst_hbm, zeros_hbm, agg_hbm,
                 src_v, dst_v, rows_v, acc_sh, sem):
    c = lax.axis_index("c")
    s = lax.axis_index("s")
    pltpu.sync_copy(zeros_hbm, acc_sh.at[pl.ds(s * RPT, RPT)])

    @pl.when(s == 0)
    def _():
        pltpu.sync_copy(
            zeros_hbm.at[pl.ds(0, TAIL)], acc_sh.at[pl.ds(NS * RPT, TAIL)]
        )

    pltpu.sync_copy(src2_hbm.at[c * NS + s], src_v)
    pltpu.sync_copy(dst_hbm.at[s], dst_v)
    plsc.subcore_barrier()

    @pl.loop(0, NCH)
    def _(j):
        pltpu.async_copy(hflat_hbm.at[src_v.at[j]], rows_v, sem).wait()
        pltpu.sync_copy(rows_v, acc_sh.at[dst_v.at[j]], add=True)

    plsc.subcore_barrier()
    pltpu.sync_copy(
        acc_sh.at[pl.ds(s * RPT, RPT)],
        agg_hbm.at[pl.ds(c * N + s * RPT, RPT)],
    )

    @pl.when(s == 0)
    def _():
        pltpu.sync_copy(
            acc_sh.at[pl.ds(NS * RPT, TAIL)],
            agg_hbm.at[pl.ds(c * N + NS * RPT, TAIL)],
        )


@functools.cache
def _build_sc_kernels():
    mesh = plsc.VectorSubcoreMesh(
        core_axis_name="c", subcore_axis_name="s", num_cores=NC, num_subcores=NS
    )
    sc_degree = pl.kernel(
        _sc_degree_body,
        out_type=jax.ShapeDtypeStruct((NC * N, HALF), jnp.float32),
        mesh=mesh,
        scratch_types=[
            pltpu.VMEM((NCH, CH), jnp.int32),
            pltpu.VMEM((CH, HALF), jnp.float32),
            pltpu.VMEM_SHARED((N, HALF), jnp.float32),
        ],
    )
    sc_agg = pl.kernel(
        _sc_agg_body,
        out_type=jax.ShapeDtypeStruct((NC * N, HALF), jnp.float32),
        mesh=mesh,
        scratch_types=[
            pltpu.VMEM((NCH, CH), jnp.int32),
            pltpu.VMEM((NCH, CH), jnp.int32),
            pltpu.VMEM((CH, HALF), jnp.float32),
            pltpu.VMEM_SHARED((N, HALF), jnp.float32),
            pltpu.SemaphoreType.DMA,
        ],
    )
    return sc_degree, sc_agg


def _dense_block(a0, a1, dg0, dg1, h0, h1, Wl, bl, Wr):
    deg = jnp.maximum(dg0[0][:, 0:1] + dg1[0][:, 0:1], 1.0)
    inv = 1.0 / deg
    return (
        jnp.dot(a0[...] * inv, Wl[0:HALF, :], preferred_element_type=jnp.float32)
        + jnp.dot(a1[...] * inv, Wl[HALF:H, :], preferred_element_type=jnp.float32)
        + jnp.dot(h0[...], Wr[0:HALF, :], preferred_element_type=jnp.float32)
        + jnp.dot(h1[...], Wr[HALF:H, :], preferred_element_type=jnp.float32)
        + bl[...]
    )


def _mm_relu_body(a0, a1, dg0, dg1, h0, h1, Wl, bl, Wr, out):
    acc = jnp.maximum(_dense_block(a0, a1, dg0, dg1, h0, h1, Wl, bl, Wr), 0.0)
    out[0] = acc[:, 0:HALF]
    out[1] = acc[:, HALF:H]


def _mm_pool_head_body(a0, a1, dg0, dg1, h0, h1, Wl, bl, Wr, batch, hW, hb,
                       out, pooled_acc, cnt_acc):
    b = pl.program_id(0)
    acc = _dense_block(a0, a1, dg0, dg1, h0, h1, Wl, bl, Wr)
    onehot = (
        batch[...] == lax.broadcasted_iota(jnp.int32, (BN, G), 1)
    ).astype(jnp.float32)

    @pl.when(b == 0)
    def _():
        pooled_acc[...] = jnp.zeros_like(pooled_acc)
        cnt_acc[...] = jnp.zeros_like(cnt_acc)

    pooled_acc[...] += lax.dot_general(
        onehot, acc, (((0,), (0,)), ((), ())),
        preferred_element_type=jnp.float32,
    )
    cnt_acc[...] += lax.dot_general(
        onehot, jnp.ones((BN, HALF), jnp.float32), (((0,), (0,)), ((), ())),
        preferred_element_type=jnp.float32,
    )

    @pl.when(b == NB - 1)
    def _():
        cnt = jnp.maximum(cnt_acc[:, 0:1], 1.0)
        pooled = pooled_acc[...] / cnt
        out[...] = (
            jnp.dot(pooled, hW[...], preferred_element_type=jnp.float32) + hb[...]
        )


def _row_spec(off):
    return pl.BlockSpec((BN, HALF), lambda b, off=off: (b + off, 0))


_common_in_specs = [
    _row_spec(0),
    _row_spec(NB),
    pl.BlockSpec((1, BN, HALF), lambda b: (0, b, 0)),
    pl.BlockSpec((1, BN, HALF), lambda b: (1, b, 0)),
    _row_spec(0),
    _row_spec(NB),
    pl.BlockSpec((H, H), lambda b: (0, 0)),
    pl.BlockSpec((1, H), lambda b: (0, 0)),
    pl.BlockSpec((H, H), lambda b: (0, 0)),
]

_mm_relu = pl.pallas_call(
    _mm_relu_body,
    grid=(NB,),
    in_specs=_common_in_specs,
    out_specs=pl.BlockSpec((2, BN, HALF), lambda b: (0, b, 0)),
    out_shape=jax.ShapeDtypeStruct((2, N, HALF), jnp.float32),
)

_mm_pool_head = pl.pallas_call(
    _mm_pool_head_body,
    grid=(NB,),
    in_specs=_common_in_specs + [
        pl.BlockSpec((BN, 1), lambda b: (b, 0)),
        pl.BlockSpec((H, HALF), lambda b: (0, 0)),
        pl.BlockSpec((1, HALF), lambda b: (0, 0)),
    ],
    out_specs=pl.BlockSpec((G, HALF), lambda b: (0, 0)),
    out_shape=jax.ShapeDtypeStruct((G, HALF), jnp.float32),
    scratch_shapes=[
        pltpu.VMEM((G, H), jnp.float32),
        pltpu.VMEM((G, HALF), jnp.float32),
    ],
)


def kernel(x, edge_index, batch, Wl0, bl0, Wr0, Wl1, bl1, Wr1, Wl2, bl2, Wr2,
           Wl3, bl3, Wr3, head_W, head_b):
    _sc_degree, _sc_agg = _build_sc_kernels()
    src = edge_index[0].astype(jnp.int32)
    dst = edge_index[1].astype(jnp.int32)
    src_rs = src.reshape(NS, NCH, CH)
    src2 = jnp.concatenate([src_rs, src_rs + N], axis=0)
    dst_rs = dst.reshape(NS, NCH, CH)

    ones_rows = jnp.ones((CH, HALF), jnp.float32)
    zeros_half = jnp.zeros((RPT, HALF), jnp.float32)

    deg2 = _sc_degree(dst_rs, ones_rows, zeros_half).reshape(NC, N, HALF)

    hflat = jnp.concatenate([x[:, 0:HALF], x[:, HALF:H]], axis=0)

    batch2d = batch.astype(jnp.int32).reshape(N, 1)
    hW_pad = jnp.pad(head_W, ((0, 0), (0, HALF - 1)))
    hb_pad = jnp.pad(head_b.reshape(1, 1), ((0, 0), (0, HALF - 1)))

    layers = ((Wl0, bl0, Wr0), (Wl1, bl1, Wr1), (Wl2, bl2, Wr2))
    for Wl, bl, Wr in layers:
        agg = _sc_agg(hflat, src2, dst_rs, zeros_half)
        hflat = _mm_relu(agg, agg, deg2, deg2, hflat, hflat,
                         Wl, bl.reshape(1, H), Wr).reshape(NC * N, HALF)

    agg = _sc_agg(hflat, src2, dst_rs, zeros_half)
    out = _mm_pool_head(agg, agg, deg2, deg2, hflat, hflat,
                        Wl3, bl3.reshape(1, H), Wr3, batch2d, hW_pad, hb_pad)
    return out[:, 0:1]


# speedup vs baseline: 7.0292x; 1.5456x over previous
import functools

import jax
import jax.numpy as jnp
from jax import lax
from jax.experimental import pallas as pl
from jax.experimental.pallas import tpu as pltpu
from jax.experimental.pallas import tpu_sc as plsc

N = 10000
E = 160000
H = 256
G = 64
HALF = 128
NC = 2
NS = 16
EPT = E // NS
CH = 80
NCH = EPT // CH
RPT = 624
TAIL = N - NS * RPT
NBUF = 2
BN = 1000
NB = N // BN

def _sc_degree_body(dst_hbm, ones_hbm, zeros_hbm, deg_hbm, dst_v, ones_v, deg_sh):
    c = lax.axis_index("c")
    s = lax.axis_index("s")
    pltpu.sync_copy(zeros_hbm, deg_sh.at[pl.ds(s * RPT, RPT)])

    @pl.when(s == 0)
    def _():
        pltpu.sync_copy(
            zeros_hbm.at[pl.ds(0, TAIL)], deg_sh.at[pl.ds(NS * RPT, TAIL)]
        )

    pltpu.sync_copy(dst_hbm.at[s], dst_v)
    pltpu.sync_copy(ones_hbm, ones_v)
    plsc.subcore_barrier()

    @pl.loop(0, NCH // 2)
    def _(j):
        pltpu.sync_copy(ones_v, deg_sh.at[dst_v.at[2 * j + c]], add=True)

    @pl.when(c == 0)
    def _():
        pltpu.sync_copy(ones_v, deg_sh.at[dst_v.at[NCH - 1]], add=True)

    plsc.subcore_barrier()
    pltpu.sync_copy(
        deg_sh.at[pl.ds(s * RPT, RPT)],
        deg_hbm.at[pl.ds(c * N + s * RPT, RPT)],
    )

    @pl.when(s == 0)
    def _():
        pltpu.sync_copy(
            deg_sh.at[pl.ds(NS * RPT, TAIL)],
            deg_hbm.at[pl.ds(c * N + NS * RPT, TAIL)],
        )


def _sc_agg_body(hflat_hbm, src2_hbm, dst_hbm, zeros_hbm, agg_hbm,
                 src_v, dst_v, rows_0, rows_1, acc_sh, sem_0, sem_1):
    c = lax.axis_index("c")
    s = lax.axis_index("s")
    rows = (rows_0, rows_1)
    sems = (sem_0, sem_1)
    pltpu.sync_copy(zeros_hbm, acc_sh.at[pl.ds(s * RPT, RPT)])

    @pl.when(s == 0)
    def _():
        pltpu.sync_copy(
            zeros_hbm.at[pl.ds(0, TAIL)], acc_sh.at[pl.ds(NS * RPT, TAIL)]
        )

    pltpu.sync_copy(src2_hbm.at[c * NS + s], src_v)
    pltpu.sync_copy(dst_hbm.at[s], dst_v)
    plsc.subcore_barrier()

    for b in range(NBUF):
        pltpu.async_copy(
            hflat_hbm.at[src_v.at[pl.ds(b * CH, CH)]], rows[b], sems[b]
        )

    @pl.loop(0, NCH // NBUF)
    def _(j):
        for b in range(NBUF):
            k = NBUF * j + b
            pltpu.make_async_copy(
                hflat_hbm.at[src_v.at[pl.ds(k * CH, CH)]], rows[b], sems[b]
            ).wait()
            pltpu.sync_copy(rows[b], acc_sh.at[dst_v.at[k]], add=True)

            @pl.when(k + NBUF < NCH)
            def _(b=b, k=k):
                pltpu.async_copy(
                    hflat_hbm.at[src_v.at[pl.ds((k + NBUF) * CH, CH)]],
                    rows[b], sems[b],
                )

    pltpu.make_async_copy(
        hflat_hbm.at[src_v.at[pl.ds((NCH - 1) * CH, CH)]], rows[0], sems[0]
    ).wait()
    pltpu.sync_copy(rows[0], acc_sh.at[dst_v.at[NCH - 1]], add=True)

    plsc.subcore_barrier()
    pltpu.sync_copy(
        acc_sh.at[pl.ds(s * RPT, RPT)],
        agg_hbm.at[pl.ds(c * N + s * RPT, RPT)],
    )

    @pl.when(s == 0)
    def _():
        pltpu.sync_copy(
            acc_sh.at[pl.ds(NS * RPT, TAIL)],
            agg_hbm.at[pl.ds(c * N + NS * RPT, TAIL)],
        )


@functools.cache
def _build_sc_kernels():
    mesh = plsc.VectorSubcoreMesh(
        core_axis_name="c", subcore_axis_name="s", num_cores=NC, num_subcores=NS
    )
    sc_degree = pl.kernel(
        _sc_degree_body,
        out_type=jax.ShapeDtypeStruct((NC * N, HALF), jnp.float32),
        mesh=mesh,
        scratch_types=[
            pltpu.VMEM((NCH, CH), jnp.int32),
            pltpu.VMEM((CH, HALF), jnp.float32),
            pltpu.VMEM_SHARED((N, HALF), jnp.float32),
        ],
    )
    sc_agg = pl.kernel(
        _sc_agg_body,
        out_type=jax.ShapeDtypeStruct((NC * N, HALF), jnp.float32),
        mesh=mesh,
        scratch_types=(
            [pltpu.VMEM((EPT,), jnp.int32), pltpu.VMEM((NCH, CH), jnp.int32)]
            + [pltpu.VMEM((CH, HALF), jnp.float32)] * NBUF
            + [pltpu.VMEM_SHARED((N, HALF), jnp.float32)]
            + [pltpu.SemaphoreType.DMA] * NBUF
        ),
    )
    return sc_degree, sc_agg


def _dense_block(a0, a1, dg0, dg1, h0, h1, Wl, bl, Wr):
    deg = jnp.maximum(dg0[0][:, 0:1] + dg1[0][:, 0:1], 1.0)
    inv = 1.0 / deg
    return (
        jnp.dot(a0[...] * inv, Wl[0:HALF, :], preferred_element_type=jnp.float32)
        + jnp.dot(a1[...] * inv, Wl[HALF:H, :], preferred_element_type=jnp.float32)
        + jnp.dot(h0[...], Wr[0:HALF, :], preferred_element_type=jnp.float32)
        + jnp.dot(h1[...], Wr[HALF:H, :], preferred_element_type=jnp.float32)
        + bl[...]
    )


def _mm_relu_body(a0, a1, dg0, dg1, h0, h1, Wl, bl, Wr, out):
    acc = jnp.maximum(_dense_block(a0, a1, dg0, dg1, h0, h1, Wl, bl, Wr), 0.0)
    out[0] = acc[:, 0:HALF]
    out[1] = acc[:, HALF:H]


def _mm_pool_head_body(a0, a1, dg0, dg1, h0, h1, Wl, bl, Wr, batch, hW, hb,
                       out, pooled_acc, cnt_acc):
    b = pl.program_id(0)
    acc = _dense_block(a0, a1, dg0, dg1, h0, h1, Wl, bl, Wr)
    onehot = (
        batch[...] == lax.broadcasted_iota(jnp.int32, (BN, G), 1)
    ).astype(jnp.float32)

    @pl.when(b == 0)
    def _():
        pooled_acc[...] = jnp.zeros_like(pooled_acc)
        cnt_acc[...] = jnp.zeros_like(cnt_acc)

    pooled_acc[...] += lax.dot_general(
        onehot, acc, (((0,), (0,)), ((), ())),
        preferred_element_type=jnp.float32,
    )
    cnt_acc[...] += lax.dot_general(
        onehot, jnp.ones((BN, HALF), jnp.float32), (((0,), (0,)), ((), ())),
        preferred_element_type=jnp.float32,
    )

    @pl.when(b == NB - 1)
    def _():
        cnt = jnp.maximum(cnt_acc[:, 0:1], 1.0)
        pooled = pooled_acc[...] / cnt
        out[...] = (
            jnp.dot(pooled, hW[...], preferred_element_type=jnp.float32) + hb[...]
        )


def _row_spec(off):
    return pl.BlockSpec((BN, HALF), lambda b, off=off: (b + off, 0))


_common_in_specs = [
    _row_spec(0),
    _row_spec(NB),
    pl.BlockSpec((1, BN, HALF), lambda b: (0, b, 0)),
    pl.BlockSpec((1, BN, HALF), lambda b: (1, b, 0)),
    _row_spec(0),
    _row_spec(NB),
    pl.BlockSpec((H, H), lambda b: (0, 0)),
    pl.BlockSpec((1, H), lambda b: (0, 0)),
    pl.BlockSpec((H, H), lambda b: (0, 0)),
]

_mm_relu = pl.pallas_call(
    _mm_relu_body,
    grid=(NB,),
    in_specs=_common_in_specs,
    out_specs=pl.BlockSpec((2, BN, HALF), lambda b: (0, b, 0)),
    out_shape=jax.ShapeDtypeStruct((2, N, HALF), jnp.float32),
)

_mm_pool_head = pl.pallas_call(
    _mm_pool_head_body,
    grid=(NB,),
    in_specs=_common_in_specs + [
        pl.BlockSpec((BN, 1), lambda b: (b, 0)),
        pl.BlockSpec((H, HALF), lambda b: (0, 0)),
        pl.BlockSpec((1, HALF), lambda b: (0, 0)),
    ],
    out_specs=pl.BlockSpec((G, HALF), lambda b: (0, 0)),
    out_shape=jax.ShapeDtypeStruct((G, HALF), jnp.float32),
    scratch_shapes=[
        pltpu.VMEM((G, H), jnp.float32),
        pltpu.VMEM((G, HALF), jnp.float32),
    ],
)


def kernel(x, edge_index, batch, Wl0, bl0, Wr0, Wl1, bl1, Wr1, Wl2, bl2, Wr2,
           Wl3, bl3, Wr3, head_W, head_b):
    _sc_degree, _sc_agg = _build_sc_kernels()
    src = edge_index[0].astype(jnp.int32)
    dst = edge_index[1].astype(jnp.int32)
    src_rs = src.reshape(NS, EPT)
    src2 = jnp.concatenate([src_rs, src_rs + N], axis=0)
    dst_rs = dst.reshape(NS, NCH, CH)

    ones_rows = jnp.ones((CH, HALF), jnp.float32)
    zeros_half = jnp.zeros((RPT, HALF), jnp.float32)

    deg2 = _sc_degree(dst_rs, ones_rows, zeros_half).reshape(NC, N, HALF)

    hflat = jnp.concatenate([x[:, 0:HALF], x[:, HALF:H]], axis=0)

    batch2d = batch.astype(jnp.int32).reshape(N, 1)
    hW_pad = jnp.pad(head_W, ((0, 0), (0, HALF - 1)))
    hb_pad = jnp.pad(head_b.reshape(1, 1), ((0, 0), (0, HALF - 1)))

    layers = ((Wl0, bl0, Wr0), (Wl1, bl1, Wr1), (Wl2, bl2, Wr2))
    for Wl, bl, Wr in layers:
        agg = _sc_agg(hflat, src2, dst_rs, zeros_half)
        hflat = _mm_relu(agg, agg, deg2, deg2, hflat, hflat,
                         Wl, bl.reshape(1, H), Wr).reshape(NC * N, HALF)

    agg = _sc_agg(hflat, src2, dst_rs, zeros_half)
    out = _mm_pool_head(agg, agg, deg2, deg2, hflat, hflat,
                        Wl3, bl3.reshape(1, H), Wr3, batch2d, hW_pad, hb_pad)
    return out[:, 0:1]
